# baseline (speedup 1.0000x reference)
import jax
import jax.numpy as jnp
from jax import lax
from jax.experimental import pallas as pl
from jax.experimental.pallas import tpu as pltpu
from jax.experimental.pallas import tpu_sc as plsc

N_SRC0 = 50000
ND1 = 25000
ND2 = 5000
D = 512
E1 = 100000
E2 = 50000

CW = 128
NCHUNK = D // CW
B = 48
ZR = 64
NSC = 16
EPG = 192


def _pad_batches(e):
    return pl.cdiv(e, NSC * EPG) * (NSC * EPG)


def _split_point(n_dst):
    return (n_dst // 2 + 127) // 128 * 128


def _make_sc_agg(n_dst, e_pad):
    nb = e_pad // B
    nbt_max = pl.cdiv(nb, NSC)
    split = _split_point(n_dst)
    sz0, sz1 = split, n_dst - split
    dump = split
    nzb = pl.cdiv(split + 1, ZR)
    acc_rows = nzb * ZR
    base0 = (sz0 // NSC) // 8 * 8
    last0 = sz0 - (NSC - 1) * base0
    base1 = (sz1 // NSC) // 8 * 8
    last1 = sz1 - (NSC - 1) * base1

    mesh = plsc.VectorSubcoreMesh(core_axis_name="c", subcore_axis_name="s")

    def body(y0, y1, y2, y3, src, dst, zc, oc,
             agg0, agg1, agg2, agg3, cnt,
             acc_sh, ewl, psrc0, pdst0, psrc1, pdst1, cntb,
             srcb0, idxv0, rowsv0, srcb1, idxv1, rowsv1, gsem0, gsem1):
        ytabs = (y0, y1, y2, y3)
        aggs = (agg0, agg1, agg2, agg3)
        sbufs = (srcb0, srcb1)
        ibufs = (idxv0, idxv1)
        rbufs = (rowsv0, rowsv1)
        sems = (gsem0, gsem1)
        cid = lax.axis_index("c")
        sid = lax.axis_index("s")
        dbase = cid * split
        locsz = jnp.where(cid == 0, sz0, sz1)

        def zero_acc():
            for i in range(pl.cdiv(nzb, NSC)):
                blk = sid + NSC * i

                @pl.when(blk < nzb)
                def _():
                    pltpu.async_copy(zc, acc_sh.at[pl.ds(blk * ZR, ZR), :],
                                     gsem0)
            for i in range(pl.cdiv(nzb, NSC)):
                blk = sid + NSC * i

                @pl.when(blk < nzb)
                def _():
                    pltpu.make_async_copy(
                        zc, acc_sh.at[pl.ds(blk * ZR, ZR), :], gsem0).wait()

        def writeout(out_ref):
            for c, bs, ls in ((0, base0, last0), (1, base1, last1)):
                @pl.when((cid == c) & (sid < NSC - 1))
                def _(bs=bs):
                    r0 = sid * bs
                    pltpu.sync_copy(acc_sh.at[pl.ds(r0, bs), :],
                                    out_ref.at[pl.ds(c * split + r0, bs), :])

                @pl.when((cid == c) & (sid == NSC - 1))
                def _(bs=bs, ls=ls):
                    r0 = (NSC - 1) * bs
                    pltpu.sync_copy(acc_sh.at[pl.ds(r0, ls), :],
                                    out_ref.at[pl.ds(c * split + r0, ls), :])

        ngrp = e_pad // (NSC * EPG)
        ebase = sid * (e_pad // NSC)
        cntb[...] = jnp.zeros((16,), jnp.int32)
        pbufs = ((psrc0, pdst0, gsem0), (psrc1, pdst1, gsem1))

        def pre_ld(g, p):
            off = ebase + g * EPG
            ps, pd, sem = pbufs[p]
            pltpu.async_copy(src.at[pl.ds(off, EPG)], ps, sem)
            pltpu.async_copy(dst.at[pl.ds(off, EPG)], pd, sem)

        def pre_proc(g, p):
            off = ebase + g * EPG
            ps, pd, sem = pbufs[p]
            pltpu.make_async_copy(src.at[pl.ds(off, EPG)], ps, sem).wait()
            pltpu.make_async_copy(dst.at[pl.ds(off, EPG)], pd, sem).wait()
            cnt_v = cntb[...]
            for t in range(EPG // 16):
                sl = pl.ds(t * 16, 16)
                l = pd[sl] - dbase
                m = (l >= 0) & (l < locsz)
                pos = cnt_v + plsc.cumsum(m.astype(jnp.int32)) - 1
                plsc.store_scatter(ewl, [pos], l * 32768 + ps[sl], mask=m)
                cnt_v = cnt_v + plsc.all_reduce_population_count(m)
            cntb[...] = cnt_v

        pre_ld(0, 0)

        def _pgrp(gp, carry):
            g0 = 2 * gp
            pre_ld(g0 + 1, 1)
            pre_proc(g0, 0)

            @pl.when(g0 + 2 < ngrp)
            def _():
                pre_ld(g0 + 2, 0)

            pre_proc(g0 + 1, 1)
            return carry

        lax.fori_loop(0, ngrp // 2, _pgrp, 0)
        if ngrp % 2 == 1:
            pre_proc(ngrp - 1, 0)

        cnt_v = cntb[...]
        iota = lax.iota(jnp.int32, 16)
        for t in range(B // 16):
            posv = cnt_v + iota + 16 * t
            plsc.store_scatter(ewl, [posv],
                               jnp.full((16,), dump * 32768, jnp.int32))
        n_mine = (cnt_v[0] + B - 1) // B

        def unpack(j, p):
            for t in range(B // 16):
                sl = pl.ds(t * 16, 16)
                w = ewl[pl.ds(j * B + t * 16, 16)]
                sbufs[p][sl] = w & 32767
                ibufs[p][sl] = w >> 15

        for k in range(NCHUNK):
            zero_acc()
            plsc.subcore_barrier()
            ytab = ytabs[k]

            def start(j, p, ytab=ytab):
                unpack(j, p)
                pltpu.async_copy(ytab.at[sbufs[p]], rbufs[p], sems[p])

            def finish_scatter(p, ytab=ytab):
                pltpu.make_async_copy(ytab.at[sbufs[p]], rbufs[p],
                                      sems[p]).wait()
                pltpu.sync_copy(rbufs[p], acc_sh.at[ibufs[p]], add=True)

            @pl.when(n_mine > 0)
            def _():
                start(0, 0)

            def _pair(ip, carry):
                j0 = 2 * ip
                start(j0 + 1, 1)
                finish_scatter(0)

                @pl.when(j0 + 2 < n_mine)
                def _():
                    start(j0 + 2, 0)

                finish_scatter(1)
                return carry

            lax.fori_loop(0, n_mine // 2, _pair, 0)

            @pl.when(n_mine % 2 == 1)
            def _():
                finish_scatter(0)

            plsc.subcore_barrier()
            writeout(aggs[k])
            plsc.subcore_barrier()

        zero_acc()
        pltpu.sync_copy(oc, rowsv0)
        plsc.subcore_barrier()

        def _cbatch(j, carry):
            unpack(j, 1)
            pltpu.sync_copy(rowsv0, acc_sh.at[idxv1], add=True)
            return carry

        lax.fori_loop(0, n_mine, _cbatch, 0)
        plsc.subcore_barrier()
        writeout(cnt)

    return pl.kernel(
        body,
        out_type=tuple(
            jax.ShapeDtypeStruct((n_dst, CW), jnp.float32)
            for _ in range(NCHUNK + 1)
        ),
        mesh=mesh,
        compiler_params=pltpu.CompilerParams(needs_layout_passes=False),
        scratch_types=[
            pltpu.VMEM_SHARED((acc_rows, CW), jnp.float32),
            pltpu.VMEM(((nbt_max + 1) * B,), jnp.int32),
            pltpu.VMEM((EPG,), jnp.int32),
            pltpu.VMEM((EPG,), jnp.int32),
            pltpu.VMEM((EPG,), jnp.int32),
            pltpu.VMEM((EPG,), jnp.int32),
            pltpu.VMEM((16,), jnp.int32),
            pltpu.VMEM((B,), jnp.int32),
            pltpu.VMEM((B,), jnp.int32),
            pltpu.VMEM((B, CW), jnp.float32),
            pltpu.VMEM((B,), jnp.int32),
            pltpu.VMEM((B,), jnp.int32),
            pltpu.VMEM((B, CW), jnp.float32),
            pltpu.SemaphoreType.DMA,
            pltpu.SemaphoreType.DMA,
        ],
    )


_sc_agg1 = _make_sc_agg(ND1, _pad_batches(E1))
_sc_agg2 = _make_sc_agg(ND2, _pad_batches(E2))


def _tc_transform(x, Wl, b, Wr, n_rows):
    R = 1000

    def body(x_ref, wl_ref, b_ref, wr_ref, y0, y1, y2, y3, z_ref):
        xb = x_ref[...].astype(jnp.bfloat16)
        y = jnp.dot(xb, wl_ref[...], preferred_element_type=jnp.float32)
        y0[...] = y[:, 0:128]
        y1[...] = y[:, 128:256]
        y2[...] = y[:, 256:384]
        y3[...] = y[:, 384:512]
        z_ref[...] = (jnp.dot(xb, wr_ref[...],
                              preferred_element_type=jnp.float32) + b_ref[...])

    yspec = pl.BlockSpec((R, CW), lambda i: (i, 0))
    return pl.pallas_call(
        body,
        grid=(n_rows // R,),
        in_specs=[
            pl.BlockSpec((R, D), lambda i: (i, 0)),
            pl.BlockSpec((D, D), lambda i: (0, 0)),
            pl.BlockSpec((1, D), lambda i: (0, 0)),
            pl.BlockSpec((D, D), lambda i: (0, 0)),
        ],
        out_specs=[yspec, yspec, yspec, yspec,
                   pl.BlockSpec((R, D), lambda i: (i, 0))],
        out_shape=[jax.ShapeDtypeStruct((n_rows, CW), jnp.float32)
                   for _ in range(4)]
                  + [jax.ShapeDtypeStruct((n_rows, D), jnp.float32)],
    )(x, Wl, b, Wr)


def _tc_combine(aggs, cnt, z, relu, n_dst):
    R = 1000

    def body(a0, a1, a2, a3, cnt_ref, z_ref, o_ref):
        inv = 1.0 / jnp.maximum(cnt_ref[:, 0:1], 1.0)
        mean = jnp.concatenate(
            [a0[...], a1[...], a2[...], a3[...]], axis=1) * inv
        acc = mean + z_ref[...]
        if relu:
            acc = jnp.maximum(acc, 0.0)
        o_ref[...] = acc

    aspec = pl.BlockSpec((R, CW), lambda i: (i, 0))
    return pl.pallas_call(
        body,
        grid=(n_dst // R,),
        in_specs=[aspec, aspec, aspec, aspec, aspec,
                  pl.BlockSpec((R, D), lambda i: (i, 0))],
        out_specs=pl.BlockSpec((R, D), lambda i: (i, 0)),
        out_shape=jax.ShapeDtypeStruct((n_dst, D), jnp.float32),
    )(*aggs, cnt, z)


def _pad_edges(ei, e_pad, dump_row):
    e = ei.shape[1]
    src = jnp.concatenate([ei[0], jnp.zeros((e_pad - e,), jnp.int32)])
    dst = jnp.concatenate([ei[1], jnp.full((e_pad - e,), dump_row, jnp.int32)])
    return src, dst


def kernel(x, edge_index1, edge_index2, W1l, b1l, W1r, W2l, b2l, W2r):
    zc = jnp.zeros((ZR, CW), jnp.float32)
    oc = jnp.ones((B, CW), jnp.float32)

    W1l16, W1r16 = W1l.astype(jnp.bfloat16), W1r.astype(jnp.bfloat16)
    W2l16, W2r16 = W2l.astype(jnp.bfloat16), W2r.astype(jnp.bfloat16)
    *y1, z1 = _tc_transform(x, W1l16, b1l.reshape(1, D), W1r16, ND1)
    src1, dst1 = _pad_edges(edge_index1, _pad_batches(E1), ND1)
    *agg1, cnt1 = _sc_agg1(*y1, src1, dst1, zc, oc)
    h = _tc_combine(agg1, cnt1, z1, True, ND1)

    *y2, z2 = _tc_transform(h, W2l16, b2l.reshape(1, D), W2r16, ND2)
    src2, dst2 = _pad_edges(edge_index2, _pad_batches(E2), ND2)
    *agg2, cnt2 = _sc_agg2(*y2, src2, dst2, zc, oc)
    out = _tc_combine(agg2, cnt2, z2, False, ND2)
    return out


# speedup vs baseline: 1.1665x; 1.1665x over previous
import jax
import jax.numpy as jnp
from jax import lax
from jax.experimental import pallas as pl
from jax.experimental.pallas import tpu as pltpu
from jax.experimental.pallas import tpu_sc as plsc

N_SRC0 = 50000
ND1 = 25000
ND2 = 5000
D = 512
E1 = 100000
E2 = 50000

CW = 128
NCHUNK = D // CW
B = 48
ZR = 64
NSC = 16
EPG = 192


def _pad_batches(e):
    return pl.cdiv(e, NSC * EPG) * (NSC * EPG)


def _split_point(n_dst):
    return (n_dst // 2 + 127) // 128 * 128


def _make_sc_agg(n_dst, e_pad, cpp):
    if cpp == 1:
        groups = [([0], False), ([1], False), ([2], False), ([3], False),
                  ([], True)]
    else:
        groups = [([0, 1], True), ([2, 3], False)]
    nreg_max = max(len(c) + (1 if wc else 0) for c, wc in groups)
    nb = e_pad // B
    nbt_max = pl.cdiv(nb, NSC)
    split = _split_point(n_dst)
    sz0, sz1 = split, n_dst - split
    dump = split
    nzb = pl.cdiv(split + 1, ZR)
    acc_rows = nzb * ZR
    base0 = (sz0 // NSC) // 8 * 8
    last0 = sz0 - (NSC - 1) * base0
    base1 = (sz1 // NSC) // 8 * 8
    last1 = sz1 - (NSC - 1) * base1

    mesh = plsc.VectorSubcoreMesh(core_axis_name="c", subcore_axis_name="s")

    def body(y0, y1, y2, y3, src, dst, zc, oc,
             agg0, agg1, agg2, agg3, cnt, *scr):
        ytabs = (y0, y1, y2, y3)
        aggs = (agg0, agg1, agg2, agg3)
        it = iter(scr)
        acc_sh, ewl = next(it), next(it)
        psrc0, pdst0, psrc1, pdst1 = (next(it) for _ in range(4))
        cntb, onesb = next(it), next(it)
        sbufs = (next(it), next(it))
        ibufs = tuple(tuple(next(it) for _ in range(2))
                      for _ in range(nreg_max))
        rbufs = tuple(tuple(next(it) for _ in range(2)) for _ in range(cpp))
        gsem0, gsem1 = next(it), next(it)
        sems = (gsem0, gsem1)
        cid = lax.axis_index("c")
        sid = lax.axis_index("s")
        dbase = cid * split
        locsz = jnp.where(cid == 0, sz0, sz1)

        def zero_acc(nreg):
            nzr = nzb * nreg
            for i in range(pl.cdiv(nzb * nreg_max, NSC)):
                blk = sid + NSC * i

                @pl.when(blk < nzr)
                def _():
                    pltpu.async_copy(zc, acc_sh.at[pl.ds(blk * ZR, ZR), :],
                                     gsem0)
            for i in range(pl.cdiv(nzb * nreg_max, NSC)):
                blk = sid + NSC * i

                @pl.when(blk < nzr)
                def _():
                    pltpu.make_async_copy(
                        zc, acc_sh.at[pl.ds(blk * ZR, ZR), :], gsem0).wait()

        def writeout(reg, out_ref):
            ro = reg * acc_rows
            for c, bs, ls in ((0, base0, last0), (1, base1, last1)):
                @pl.when((cid == c) & (sid < NSC - 1))
                def _(bs=bs):
                    r0 = sid * bs
                    pltpu.sync_copy(acc_sh.at[pl.ds(ro + r0, bs), :],
                                    out_ref.at[pl.ds(c * split + r0, bs), :])

                @pl.when((cid == c) & (sid == NSC - 1))
                def _(bs=bs, ls=ls):
                    r0 = (NSC - 1) * bs
                    pltpu.sync_copy(acc_sh.at[pl.ds(ro + r0, ls), :],
                                    out_ref.at[pl.ds(c * split + r0, ls), :])

        ngrp = e_pad // (NSC * EPG)
        ebase = sid * (e_pad // NSC)
        cntb[...] = jnp.zeros((16,), jnp.int32)
        pbufs = ((psrc0, pdst0, gsem0), (psrc1, pdst1, gsem1))

        def pre_ld(g, p):
            off = ebase + g * EPG
            ps, pd, sem = pbufs[p]
            pltpu.async_copy(src.at[pl.ds(off, EPG)], ps, sem)
            pltpu.async_copy(dst.at[pl.ds(off, EPG)], pd, sem)

        def pre_proc(g, p):
            off = ebase + g * EPG
            ps, pd, sem = pbufs[p]
            pltpu.make_async_copy(src.at[pl.ds(off, EPG)], ps, sem).wait()
            pltpu.make_async_copy(dst.at[pl.ds(off, EPG)], pd, sem).wait()
            cnt_v = cntb[...]
            for t in range(EPG // 16):
                sl = pl.ds(t * 16, 16)
                l = pd[sl] - dbase
                m = (l >= 0) & (l < locsz)
                pos = cnt_v + plsc.cumsum(m.astype(jnp.int32)) - 1
                plsc.store_scatter(ewl, [pos], l * 32768 + ps[sl], mask=m)
                cnt_v = cnt_v + plsc.all_reduce_population_count(m)
            cntb[...] = cnt_v

        pre_ld(0, 0)

        def _pgrp(gp, carry):
            g0 = 2 * gp
            pre_ld(g0 + 1, 1)
            pre_proc(g0, 0)

            @pl.when(g0 + 2 < ngrp)
            def _():
                pre_ld(g0 + 2, 0)

            pre_proc(g0 + 1, 1)
            return carry

        lax.fori_loop(0, ngrp // 2, _pgrp, 0)
        if ngrp % 2 == 1:
            pre_proc(ngrp - 1, 0)

        cnt_v = cntb[...]
        iota = lax.iota(jnp.int32, 16)
        for t in range(B // 16):
            posv = cnt_v + iota + 16 * t
            plsc.store_scatter(ewl, [posv],
                               jnp.full((16,), dump * 32768, jnp.int32))
        n_mine = (cnt_v[0] + B - 1) // B

        def unpack(j, p):
            for t in range(B // 16):
                sl = pl.ds(t * 16, 16)
                w = ewl[pl.ds(j * B + t * 16, 16)]
                sbufs[p][sl] = w & 32767
                ix = w >> 15
                for r in range(nreg_max):
                    ibufs[r][p][sl] = ix + r * acc_rows

        pltpu.sync_copy(oc, onesb)

        for chs, wc in groups:
            zero_acc(len(chs) + (1 if wc else 0))
            plsc.subcore_barrier()

            def start(j, p, chs=chs):
                unpack(j, p)
                for i, k in enumerate(chs):
                    pltpu.async_copy(ytabs[k].at[sbufs[p]], rbufs[i][p],
                                     sems[p])

            def finish_scatter(p, chs=chs, wc=wc):
                for i, k in enumerate(chs):
                    pltpu.make_async_copy(ytabs[k].at[sbufs[p]],
                                          rbufs[i][p], sems[p]).wait()
                    pltpu.sync_copy(rbufs[i][p], acc_sh.at[ibufs[i][p]],
                                    add=True)
                if wc:
                    pltpu.sync_copy(onesb, acc_sh.at[ibufs[len(chs)][p]],
                                    add=True)

            @pl.when(n_mine > 0)
            def _():
                start(0, 0)

            def _pair(ip, carry):
                j0 = 2 * ip
                start(j0 + 1, 1)
                finish_scatter(0)

                @pl.when(j0 + 2 < n_mine)
                def _():
                    start(j0 + 2, 0)

                finish_scatter(1)
                return carry

            lax.fori_loop(0, n_mine // 2, _pair, 0)

            @pl.when(n_mine % 2 == 1)
            def _():
                finish_scatter(0)

            plsc.subcore_barrier()
            for i, k in enumerate(chs):
                writeout(i, aggs[k])
            if wc:
                writeout(len(chs), cnt)
            plsc.subcore_barrier()

    return pl.kernel(
        body,
        out_type=tuple(
            jax.ShapeDtypeStruct((n_dst, CW), jnp.float32)
            for _ in range(NCHUNK + 1)
        ),
        mesh=mesh,
        compiler_params=pltpu.CompilerParams(needs_layout_passes=False),
        scratch_types=(
            [pltpu.VMEM_SHARED((nreg_max * acc_rows, CW), jnp.float32),
             pltpu.VMEM(((nbt_max + 1) * B,), jnp.int32)]
            + [pltpu.VMEM((EPG,), jnp.int32) for _ in range(4)]
            + [pltpu.VMEM((16,), jnp.int32),
               pltpu.VMEM((B, CW), jnp.float32),
               pltpu.VMEM((B,), jnp.int32),
               pltpu.VMEM((B,), jnp.int32)]
            + [pltpu.VMEM((B,), jnp.int32) for _ in range(2 * nreg_max)]
            + [pltpu.VMEM((B, CW), jnp.float32) for _ in range(2 * cpp)]
            + [pltpu.SemaphoreType.DMA, pltpu.SemaphoreType.DMA]
        ),
    )


_sc_agg1 = _make_sc_agg(ND1, _pad_batches(E1), 1)
_sc_agg2 = _make_sc_agg(ND2, _pad_batches(E2), 2)


def _tc_transform(x, Wl, b, Wr, n_rows):
    R = 1000

    def body(x_ref, wl_ref, b_ref, wr_ref, y0, y1, y2, y3, z_ref):
        xb = x_ref[...]
        y = jnp.dot(xb, wl_ref[...], preferred_element_type=jnp.float32)
        y0[...] = y[:, 0:128]
        y1[...] = y[:, 128:256]
        y2[...] = y[:, 256:384]
        y3[...] = y[:, 384:512]
        z_ref[...] = (jnp.dot(xb, wr_ref[...],
                              preferred_element_type=jnp.float32) + b_ref[...])

    yspec = pl.BlockSpec((R, CW), lambda i: (i, 0))
    return pl.pallas_call(
        body,
        grid=(n_rows // R,),
        in_specs=[
            pl.BlockSpec((R, D), lambda i: (i, 0)),
            pl.BlockSpec((D, D), lambda i: (0, 0)),
            pl.BlockSpec((1, D), lambda i: (0, 0)),
            pl.BlockSpec((D, D), lambda i: (0, 0)),
        ],
        out_specs=[yspec, yspec, yspec, yspec,
                   pl.BlockSpec((R, D), lambda i: (i, 0))],
        out_shape=[jax.ShapeDtypeStruct((n_rows, CW), jnp.float32)
                   for _ in range(4)]
                  + [jax.ShapeDtypeStruct((n_rows, D), jnp.float32)],
    )(x, Wl, b, Wr)


def _tc_combine(aggs, cnt, z, relu, n_dst):
    R = 1000

    def body(a0, a1, a2, a3, cnt_ref, z_ref, o_ref):
        inv = 1.0 / jnp.maximum(cnt_ref[:, 0:1], 1.0)
        mean = jnp.concatenate(
            [a0[...], a1[...], a2[...], a3[...]], axis=1) * inv
        acc = mean + z_ref[...]
        if relu:
            acc = jnp.maximum(acc, 0.0)
        o_ref[...] = acc

    aspec = pl.BlockSpec((R, CW), lambda i: (i, 0))
    return pl.pallas_call(
        body,
        grid=(n_dst // R,),
        in_specs=[aspec, aspec, aspec, aspec, aspec,
                  pl.BlockSpec((R, D), lambda i: (i, 0))],
        out_specs=pl.BlockSpec((R, D), lambda i: (i, 0)),
        out_shape=jax.ShapeDtypeStruct((n_dst, D), jnp.float32),
    )(*aggs, cnt, z)


def _pad_edges(ei, e_pad, dump_row):
    e = ei.shape[1]
    src = jnp.concatenate([ei[0], jnp.zeros((e_pad - e,), jnp.int32)])
    dst = jnp.concatenate([ei[1], jnp.full((e_pad - e,), dump_row, jnp.int32)])
    return src, dst


def kernel(x, edge_index1, edge_index2, W1l, b1l, W1r, W2l, b2l, W2r):
    zc = jnp.zeros((ZR, CW), jnp.float32)
    oc = jnp.ones((B, CW), jnp.float32)

    *y1, z1 = _tc_transform(x, W1l, b1l.reshape(1, D), W1r, ND1)
    src1, dst1 = _pad_edges(edge_index1, _pad_batches(E1), ND1)
    *agg1, cnt1 = _sc_agg1(*y1, src1, dst1, zc, oc)
    h = _tc_combine(agg1, cnt1, z1, True, ND1)

    *y2, z2 = _tc_transform(h, W2l, b2l.reshape(1, D), W2r, ND2)
    src2, dst2 = _pad_edges(edge_index2, _pad_batches(E2), ND2)
    *agg2, cnt2 = _sc_agg2(*y2, src2, dst2, zc, oc)
    out = _tc_combine(agg2, cnt2, z2, False, ND2)
    return out
